# baseline (speedup 1.0000x reference)
import dataclasses
import functools

import jax
import jax.numpy as jnp
from jax import lax
from jax.experimental import pallas as pl
from jax.experimental.pallas import tpu as pltpu
from jax.experimental.pallas import tpu_sc as plsc

N0, N1, N2 = 10000, 4096, 1024
E0, E1 = 320000, 131072
D = 128
NC, NS = 2, 16
NW = NC * NS
C = 128
NB = 4
PAD = 128

NCH0 = 80
NCH1 = 32


def _make_sc_agg(n_tgt, n_chunks):
    ntp = n_tgt + PAD
    per_w = n_chunks * C
    rows_per = ntp // NS
    assert rows_per * NS == ntp and rows_per % 8 == 0
    assert n_chunks % NB == 0 and n_chunks >= 2 * NB
    mesh = plsc.VectorSubcoreMesh(core_axis_name="c", subcore_axis_name="s")
    cp = pltpu.CompilerParams()
    if "needs_layout_passes" in pltpu.CompilerParams.__dataclass_fields__:
        cp = dataclasses.replace(cp, needs_layout_passes=False)

    @functools.partial(
        pl.kernel,
        compiler_params=cp,
        out_type=(jax.ShapeDtypeStruct((NC, ntp, D), jnp.float32),
                  jax.ShapeDtypeStruct((NW * ntp,), jnp.float32)),
        mesh=mesh,
        scratch_types=[
            pltpu.VMEM((per_w,), jnp.int32),
            pltpu.VMEM((n_chunks, C), jnp.int32),
            pltpu.VMEM((C, D), jnp.float32),
            pltpu.VMEM((C, D), jnp.float32),
            pltpu.VMEM((C, D), jnp.float32),
            pltpu.VMEM((C, D), jnp.float32),
            pltpu.VMEM((C,), jnp.float32),
            pltpu.VMEM((ntp,), jnp.float32),
            pltpu.VMEM_SHARED((ntp, D), jnp.float32),
        ] + [pltpu.SemaphoreType.DMA] * (2 * NB),
    )
    def agg(x_hbm, src_hbm, dst_hbm, zs_hbm, zc_hbm, on_hbm,
            sum_hbm, cnt_hbm,
            src_v, dst_v, r0_v, r1_v, r2_v, r3_v, on_v, hist_v, acc_sh,
            *sems):
        rows = (r0_v, r1_v, r2_v, r3_v)
        gsem = sems[0:NB]
        ssem = sems[NB:2 * NB]
        c = lax.axis_index("c")
        s = lax.axis_index("s")
        wid = c * NS + s

        pltpu.sync_copy(src_hbm.at[pl.ds(wid * per_w, per_w)], src_v)
        pltpu.sync_copy(dst_hbm.at[pl.ds(wid * n_chunks, n_chunks)], dst_v)
        pltpu.sync_copy(on_hbm, on_v)
        pltpu.sync_copy(zc_hbm, hist_v)
        r0 = s * rows_per
        pltpu.sync_copy(zs_hbm.at[pl.ds(r0, rows_per)],
                        acc_sh.at[pl.ds(r0, rows_per)])
        plsc.subcore_barrier()
        ones16 = on_v[pl.ds(0, 16)]

        def issue_gather(j, b):
            pltpu.async_copy(x_hbm.at[src_v.at[pl.ds(j * C, C)]],
                             rows[b], gsem[b])

        def wait_gather(j, b):
            pltpu.make_async_copy(x_hbm.at[src_v.at[pl.ds(j * C, C)]],
                                  rows[b], gsem[b]).wait()

        def issue_scatter(j, b):
            pltpu.async_copy(rows[b], acc_sh.at[dst_v.at[j]], ssem[b],
                             add=True)

        def wait_scatter(j, b):
            pltpu.make_async_copy(rows[b], acc_sh.at[dst_v.at[j]],
                                  ssem[b]).wait()

        def count_chunk(j):
            for i in range(C // 16):
                idx = dst_v[j, pl.ds(i * 16, 16)]
                plsc.addupdate_scatter(hist_v, [idx], ones16)

        issue_gather(0, 0)
        issue_gather(1, 1)

        for b in range(NB):
            j = b
            if j >= 2:
                wait_scatter(j - 2, (b + 2) % NB)
            issue_gather(j + 2, (b + 2) % NB)
            count_chunk(j)
            wait_gather(j, b)
            issue_scatter(j, b)

        @pl.loop(1, n_chunks // NB - 1)
        def _(k):
            j0 = k * NB
            for b in range(NB):
                j = j0 + b
                wait_scatter(j - 2, (b + 2) % NB)
                issue_gather(j + 2, (b + 2) % NB)
                count_chunk(j)
                wait_gather(j, b)
                issue_scatter(j, b)

        for b in range(NB):
            j = n_chunks - NB + b
            if b < 2:
                wait_scatter(j - 2, (b + 2) % NB)
                issue_gather(j + 2, (b + 2) % NB)
            count_chunk(j)
            wait_gather(j, b)
            issue_scatter(j, b)
        for b in range(NB):
            wait_scatter(n_chunks - NB + b, b)

        plsc.subcore_barrier()
        pltpu.sync_copy(acc_sh.at[pl.ds(r0, rows_per)],
                        sum_hbm.at[c, pl.ds(r0, rows_per)])
        pltpu.sync_copy(hist_v, cnt_hbm.at[pl.ds(wid * ntp, ntp)])

    return agg


def _right_mm(n_tgt, x, wr, b):
    def body(x_ref, wr_ref, b_ref, o_ref):
        o_ref[...] = lax.dot_general(
            x_ref[...], wr_ref[...], (((1,), (1,)), ((), ())),
            preferred_element_type=jnp.float32,
            precision=lax.Precision.HIGHEST) + b_ref[...]
    return pl.pallas_call(
        body,
        in_specs=[pl.BlockSpec((n_tgt, D), lambda i: (0, 0)),
                  pl.BlockSpec((D, D), lambda i: (0, 0)),
                  pl.BlockSpec((1, D), lambda i: (0, 0))],
        out_specs=pl.BlockSpec((n_tgt, D), lambda i: (0, 0)),
        out_shape=jax.ShapeDtypeStruct((n_tgt, D), jnp.float32),
        grid=(1,),
    )(x, wr, b.reshape(1, D))


def _dense_body(relu, logsm):
    def body(s_ref, c_ref, xr_ref, wl_ref, o_ref):
        ssum = s_ref[0] + s_ref[1]
        cnt_bcast = lax.dot_general(
            c_ref[...], jnp.ones((NW, D), jnp.float32),
            (((0,), (0,)), ((), ())),
            preferred_element_type=jnp.float32,
            precision=lax.Precision.HIGHEST)
        mean = ssum / jnp.maximum(cnt_bcast, 1.0)
        z = lax.dot_general(mean, wl_ref[...], (((1,), (1,)), ((), ())),
                            preferred_element_type=jnp.float32,
                            precision=lax.Precision.HIGHEST) + xr_ref[...]
        if relu:
            z = jnp.maximum(z, 0.0)
        if logsm:
            m = jnp.max(z, axis=-1, keepdims=True)
            z = z - m - jnp.log(jnp.sum(jnp.exp(z - m), axis=-1, keepdims=True))
        o_ref[...] = z
    return body


def _dense(relu, logsm, n_tgt, sums, cnts, xr, wl):
    ntp = n_tgt + PAD
    return pl.pallas_call(
        _dense_body(relu, logsm),
        in_specs=[pl.BlockSpec((NC, n_tgt, D), lambda i: (0, 0, 0)),
                  pl.BlockSpec((NW, n_tgt), lambda i: (0, 0)),
                  pl.BlockSpec((n_tgt, D), lambda i: (0, 0)),
                  pl.BlockSpec((D, D), lambda i: (0, 0))],
        out_specs=pl.BlockSpec((n_tgt, D), lambda i: (0, 0)),
        out_shape=jax.ShapeDtypeStruct((n_tgt, D), jnp.float32),
        grid=(1,),
    )(sums, cnts.reshape(NW, ntp), xr, wl)


_agg0 = _make_sc_agg(N1, NCH0)
_agg1 = _make_sc_agg(N2, NCH1)

def _pad_edges(src, dst, n_edges_pad, sentinel):
    pad = n_edges_pad - src.shape[0]
    if pad:
        i = jnp.arange(pad, dtype=jnp.int32)
        src = jnp.concatenate([src, i % jnp.int32(sentinel)])
        dst = jnp.concatenate([dst, sentinel + i % jnp.int32(PAD)])
    return src, dst.reshape(-1, C)


def kernel(x, edge_index0, edge_index1, W_l0, W_r0, b0, W_l1, W_r1, b1):
    src0 = edge_index0[0].astype(jnp.int32)
    dst0 = edge_index0[1].astype(jnp.int32)
    src1 = edge_index1[0].astype(jnp.int32)
    dst1 = edge_index1[1].astype(jnp.int32)

    src0, dst0 = _pad_edges(src0, dst0, NW * NCH0 * C, N1)
    src1, dst1 = _pad_edges(src1, dst1, NW * NCH1 * C, N2)
    ones_c = jnp.ones((C,), jnp.float32)

    zs0 = jnp.zeros((N1 + PAD, D), jnp.float32)
    zc0 = jnp.zeros((N1 + PAD,), jnp.float32)
    xr0 = _right_mm(N1, x, W_r0, b0)
    sums0, cnts0 = _agg0(x, src0, dst0, zs0, zc0, ones_c)
    h = _dense(True, False, N1, sums0, cnts0, xr0, W_l0)

    zs1 = jnp.zeros((N2 + PAD, D), jnp.float32)
    zc1 = jnp.zeros((N2 + PAD,), jnp.float32)
    xr1 = _right_mm(N2, h, W_r1, b1)
    sums1, cnts1 = _agg1(h, src1, dst1, zs1, zc1, ones_c)
    out = _dense(False, True, N2, sums1, cnts1, xr1, W_l1)
    return out


# speedup vs baseline: 1.0213x; 1.0213x over previous
import dataclasses
import functools

import jax
import jax.numpy as jnp
from jax import lax
from jax.experimental import pallas as pl
from jax.experimental.pallas import tpu as pltpu
from jax.experimental.pallas import tpu_sc as plsc

N0, N1, N2 = 10000, 4096, 1024
E0, E1 = 320000, 131072
D = 128
NC, NS = 2, 16
NW = NC * NS
C = 128
NB = 4
PAD = 128

NCH0 = 80
NCH1 = 32


def _make_sc_agg(n_tgt, n_chunks):
    ntp = n_tgt + PAD
    per_w = n_chunks * C
    rows_per = ntp // NS
    assert rows_per * NS == ntp and rows_per % 8 == 0
    assert n_chunks % NB == 0 and n_chunks >= 2 * NB
    mesh = plsc.VectorSubcoreMesh(core_axis_name="c", subcore_axis_name="s")
    cp = pltpu.CompilerParams()
    if "needs_layout_passes" in pltpu.CompilerParams.__dataclass_fields__:
        cp = dataclasses.replace(cp, needs_layout_passes=False)

    @functools.partial(
        pl.kernel,
        compiler_params=cp,
        out_type=(jax.ShapeDtypeStruct((NC, ntp, D), jnp.float32),
                  jax.ShapeDtypeStruct((NC * ntp,), jnp.float32)),
        mesh=mesh,
        scratch_types=[
            pltpu.VMEM((per_w,), jnp.int32),
            pltpu.VMEM((n_chunks, C), jnp.int32),
            pltpu.VMEM((C, D), jnp.float32),
            pltpu.VMEM((C, D), jnp.float32),
            pltpu.VMEM((C, D), jnp.float32),
            pltpu.VMEM((C, D), jnp.float32),
            pltpu.VMEM((C,), jnp.float32),
            pltpu.VMEM((rows_per,), jnp.float32),
            pltpu.VMEM_SHARED((ntp, D), jnp.float32),
            pltpu.VMEM_SHARED((ntp,), jnp.float32),
        ] + [pltpu.SemaphoreType.DMA] * (3 * NB),
    )
    def agg(x_hbm, src_hbm, dst_hbm, zs_hbm, zc_hbm, on_hbm,
            sum_hbm, cnt_hbm,
            src_v, dst_v, r0_v, r1_v, r2_v, r3_v, on_v, zst_v, acc_sh, cnt_sh,
            *sems):
        rows = (r0_v, r1_v, r2_v, r3_v)
        gsem = sems[0:NB]
        ssem = sems[NB:2 * NB]
        csem = sems[2 * NB:3 * NB]
        c = lax.axis_index("c")
        s = lax.axis_index("s")
        wid = c * NS + s

        pltpu.sync_copy(src_hbm.at[pl.ds(wid * per_w, per_w)], src_v)
        pltpu.sync_copy(dst_hbm.at[pl.ds(wid * n_chunks, n_chunks)], dst_v)
        pltpu.sync_copy(on_hbm, on_v)
        r0 = s * rows_per
        pltpu.sync_copy(zs_hbm.at[pl.ds(r0, rows_per)],
                        acc_sh.at[pl.ds(r0, rows_per)])
        pltpu.sync_copy(zc_hbm.at[pl.ds(r0, rows_per)], zst_v)
        pltpu.sync_copy(zst_v, cnt_sh.at[pl.ds(r0, rows_per)])
        plsc.subcore_barrier()

        def issue_gather(j, b):
            pltpu.async_copy(x_hbm.at[src_v.at[pl.ds(j * C, C)]],
                             rows[b], gsem[b])

        def wait_gather(j, b):
            pltpu.make_async_copy(x_hbm.at[src_v.at[pl.ds(j * C, C)]],
                                  rows[b], gsem[b]).wait()

        def issue_scatter(j, b):
            pltpu.async_copy(rows[b], acc_sh.at[dst_v.at[j]], ssem[b],
                             add=True)
            pltpu.async_copy(on_v, cnt_sh.at[dst_v.at[j]], csem[b],
                             add=True)

        def wait_scatter(j, b):
            pltpu.make_async_copy(rows[b], acc_sh.at[dst_v.at[j]],
                                  ssem[b]).wait()
            pltpu.make_async_copy(on_v, cnt_sh.at[dst_v.at[j]],
                                  csem[b]).wait()

        issue_gather(0, 0)
        issue_gather(1, 1)

        for b in range(NB):
            j = b
            if j >= 2:
                wait_scatter(j - 2, (b + 2) % NB)
            issue_gather(j + 2, (b + 2) % NB)
            wait_gather(j, b)
            issue_scatter(j, b)

        @pl.loop(1, n_chunks // NB - 1)
        def _(k):
            j0 = k * NB
            for b in range(NB):
                j = j0 + b
                wait_scatter(j - 2, (b + 2) % NB)
                issue_gather(j + 2, (b + 2) % NB)
                wait_gather(j, b)
                issue_scatter(j, b)

        for b in range(NB):
            j = n_chunks - NB + b
            if b < 2:
                wait_scatter(j - 2, (b + 2) % NB)
                issue_gather(j + 2, (b + 2) % NB)
            wait_gather(j, b)
            issue_scatter(j, b)
        for b in range(NB):
            wait_scatter(n_chunks - NB + b, b)

        plsc.subcore_barrier()
        pltpu.sync_copy(acc_sh.at[pl.ds(r0, rows_per)],
                        sum_hbm.at[c, pl.ds(r0, rows_per)])
        pltpu.sync_copy(cnt_sh.at[pl.ds(r0, rows_per)], zst_v)
        pltpu.sync_copy(zst_v, cnt_hbm.at[pl.ds(c * ntp + r0, rows_per)])

    return agg


def _dense_body(relu, logsm):
    def body(s_ref, c_ref, xt_ref, wl_ref, wr_ref, b_ref, o_ref):
        ssum = s_ref[0] + s_ref[1]
        cnt_bcast = lax.dot_general(
            c_ref[...], jnp.ones((NC, D), jnp.float32),
            (((0,), (0,)), ((), ())),
            preferred_element_type=jnp.float32,
            precision=lax.Precision.HIGHEST)
        mean = ssum / jnp.maximum(cnt_bcast, 1.0)
        z = lax.dot_general(mean, wl_ref[...], (((1,), (1,)), ((), ())),
                            preferred_element_type=jnp.float32,
                            precision=lax.Precision.HIGHEST)
        z += lax.dot_general(xt_ref[...], wr_ref[...], (((1,), (1,)), ((), ())),
                             preferred_element_type=jnp.float32,
                             precision=lax.Precision.HIGHEST)
        z += b_ref[...]
        if relu:
            z = jnp.maximum(z, 0.0)
        if logsm:
            m = jnp.max(z, axis=-1, keepdims=True)
            z = z - m - jnp.log(jnp.sum(jnp.exp(z - m), axis=-1, keepdims=True))
        o_ref[...] = z
    return body


def _dense(relu, logsm, n_tgt, sums, cnts, xt, wl, wr, b):
    ntp = n_tgt + PAD
    return pl.pallas_call(
        _dense_body(relu, logsm),
        in_specs=[pl.BlockSpec((NC, n_tgt, D), lambda i: (0, 0, 0)),
                  pl.BlockSpec((NC, n_tgt), lambda i: (0, 0)),
                  pl.BlockSpec((n_tgt, D), lambda i: (0, 0)),
                  pl.BlockSpec((D, D), lambda i: (0, 0)),
                  pl.BlockSpec((D, D), lambda i: (0, 0)),
                  pl.BlockSpec((1, D), lambda i: (0, 0))],
        out_specs=pl.BlockSpec((n_tgt, D), lambda i: (0, 0)),
        out_shape=jax.ShapeDtypeStruct((n_tgt, D), jnp.float32),
        grid=(1,),
    )(sums, cnts.reshape(NC, ntp), xt, wl, wr, b.reshape(1, D))


_agg0 = _make_sc_agg(N1, NCH0)
_agg1 = _make_sc_agg(N2, NCH1)

def _pad_edges(src, dst, n_edges_pad, sentinel):
    pad = n_edges_pad - src.shape[0]
    if pad:
        i = jnp.arange(pad, dtype=jnp.int32)
        src = jnp.concatenate([src, i % jnp.int32(sentinel)])
        dst = jnp.concatenate([dst, sentinel + i % jnp.int32(PAD)])
    return src, dst.reshape(-1, C)


def kernel(x, edge_index0, edge_index1, W_l0, W_r0, b0, W_l1, W_r1, b1):
    src0 = edge_index0[0].astype(jnp.int32)
    dst0 = edge_index0[1].astype(jnp.int32)
    src1 = edge_index1[0].astype(jnp.int32)
    dst1 = edge_index1[1].astype(jnp.int32)

    src0, dst0 = _pad_edges(src0, dst0, NW * NCH0 * C, N1)
    src1, dst1 = _pad_edges(src1, dst1, NW * NCH1 * C, N2)
    ones_c = jnp.ones((C,), jnp.float32)

    zs0 = jnp.zeros((N1 + PAD, D), jnp.float32)
    zc0 = jnp.zeros((N1 + PAD,), jnp.float32)
    sums0, cnts0 = _agg0(x, src0, dst0, zs0, zc0, ones_c)
    h = _dense(True, False, N1, sums0, cnts0, x, W_l0, W_r0, b0)

    zs1 = jnp.zeros((N2 + PAD, D), jnp.float32)
    zc1 = jnp.zeros((N2 + PAD,), jnp.float32)
    sums1, cnts1 = _agg1(h, src1, dst1, zs1, zc1, ones_c)
    out = _dense(False, True, N2, sums1, cnts1, h, W_l1, W_r1, b1)
    return out


# speedup vs baseline: 1.0579x; 1.0359x over previous
import dataclasses
import functools

import jax
import jax.numpy as jnp
from jax import lax
from jax.experimental import pallas as pl
from jax.experimental.pallas import tpu as pltpu
from jax.experimental.pallas import tpu_sc as plsc

N0, N1, N2 = 10000, 4096, 1024
E0, E1 = 320000, 131072
D = 128
NC, NS = 2, 16
NW = NC * NS
C = 128
NB = 4
PAD = 128

NCH0 = 80
NCH1 = 32


def _make_sc_agg(n_tgt, n_chunks):
    ntp = n_tgt + PAD
    per_w = n_chunks * C
    rows_per = ntp // NS
    assert rows_per * NS == ntp and rows_per % 8 == 0
    assert n_chunks % NB == 0 and n_chunks >= 2 * NB
    mesh = plsc.VectorSubcoreMesh(core_axis_name="c", subcore_axis_name="s")
    cp = pltpu.CompilerParams()
    if "needs_layout_passes" in pltpu.CompilerParams.__dataclass_fields__:
        cp = dataclasses.replace(cp, needs_layout_passes=False)

    @functools.partial(
        pl.kernel,
        compiler_params=cp,
        out_type=(jax.ShapeDtypeStruct((NC, ntp, D), jnp.float32),
                  jax.ShapeDtypeStruct((NC * ntp,), jnp.float32)),
        mesh=mesh,
        scratch_types=[
            pltpu.VMEM((per_w,), jnp.int32),
            pltpu.VMEM((n_chunks, C), jnp.int32),
            pltpu.VMEM((C, D), jnp.float32),
            pltpu.VMEM((C, D), jnp.float32),
            pltpu.VMEM((C, D), jnp.float32),
            pltpu.VMEM((C, D), jnp.float32),
            pltpu.VMEM((C,), jnp.float32),
            pltpu.VMEM((rows_per,), jnp.float32),
            pltpu.VMEM_SHARED((ntp, D), jnp.float32),
            pltpu.VMEM_SHARED((ntp,), jnp.float32),
        ] + [pltpu.SemaphoreType.DMA] * (3 * NB),
    )
    def agg(x_hbm, src_hbm, dst_hbm, zs_hbm, zc_hbm, on_hbm,
            sum_hbm, cnt_hbm,
            src_v, dst_v, r0_v, r1_v, r2_v, r3_v, on_v, zst_v, acc_sh, cnt_sh,
            *sems):
        rows = (r0_v, r1_v, r2_v, r3_v)
        gsem = sems[0:NB]
        ssem = sems[NB:2 * NB]
        csem = sems[2 * NB:3 * NB]
        c = lax.axis_index("c")
        s = lax.axis_index("s")
        wid = c * NS + s

        def issue_gather(j, b):
            pltpu.async_copy(x_hbm.at[src_v.at[pl.ds(j * C, C)]],
                             rows[b], gsem[b])

        def wait_gather(j, b):
            pltpu.make_async_copy(x_hbm.at[src_v.at[pl.ds(j * C, C)]],
                                  rows[b], gsem[b]).wait()

        r0 = s * rows_per
        pltpu.async_copy(src_hbm.at[pl.ds(wid * per_w, per_w)], src_v,
                         gsem[0])
        pltpu.async_copy(dst_hbm.at[pl.ds(wid * n_chunks, n_chunks)], dst_v,
                         gsem[1])
        pltpu.async_copy(on_hbm, on_v, csem[0])
        pltpu.async_copy(zs_hbm.at[pl.ds(r0, rows_per)],
                         acc_sh.at[pl.ds(r0, rows_per)], ssem[0])
        pltpu.make_async_copy(src_hbm.at[pl.ds(wid * per_w, per_w)], src_v,
                              gsem[0]).wait()
        issue_gather(0, 0)
        issue_gather(1, 1)
        pltpu.make_async_copy(dst_hbm.at[pl.ds(wid * n_chunks, n_chunks)],
                              dst_v, gsem[1]).wait()
        pltpu.make_async_copy(on_hbm, on_v, csem[0]).wait()
        pltpu.make_async_copy(zs_hbm.at[pl.ds(r0, rows_per)],
                              acc_sh.at[pl.ds(r0, rows_per)], ssem[0]).wait()
        pltpu.sync_copy(zc_hbm.at[pl.ds(r0, rows_per)], zst_v)
        pltpu.sync_copy(zst_v, cnt_sh.at[pl.ds(r0, rows_per)])
        plsc.subcore_barrier()

        def issue_scatter(j, b):
            pltpu.async_copy(rows[b], acc_sh.at[dst_v.at[j]], ssem[b],
                             add=True)
            pltpu.async_copy(on_v, cnt_sh.at[dst_v.at[j]], csem[b],
                             add=True)

        def wait_scatter(j, b):
            pltpu.make_async_copy(rows[b], acc_sh.at[dst_v.at[j]],
                                  ssem[b]).wait()
            pltpu.make_async_copy(on_v, cnt_sh.at[dst_v.at[j]],
                                  csem[b]).wait()


        for b in range(NB):
            j = b
            if j >= 2:
                wait_scatter(j - 2, (b + 2) % NB)
            issue_gather(j + 2, (b + 2) % NB)
            wait_gather(j, b)
            issue_scatter(j, b)

        @pl.loop(1, n_chunks // NB - 1)
        def _(k):
            j0 = k * NB
            for b in range(NB):
                j = j0 + b
                wait_scatter(j - 2, (b + 2) % NB)
                issue_gather(j + 2, (b + 2) % NB)
                wait_gather(j, b)
                issue_scatter(j, b)

        for b in range(NB):
            j = n_chunks - NB + b
            if b < 2:
                wait_scatter(j - 2, (b + 2) % NB)
                issue_gather(j + 2, (b + 2) % NB)
            wait_gather(j, b)
            issue_scatter(j, b)
        for b in range(NB):
            wait_scatter(n_chunks - NB + b, b)

        plsc.subcore_barrier()
        pltpu.async_copy(acc_sh.at[pl.ds(r0, rows_per)],
                         sum_hbm.at[c, pl.ds(r0, rows_per)], ssem[0])
        pltpu.sync_copy(cnt_sh.at[pl.ds(r0, rows_per)], zst_v)
        pltpu.sync_copy(zst_v, cnt_hbm.at[pl.ds(c * ntp + r0, rows_per)])
        pltpu.make_async_copy(acc_sh.at[pl.ds(r0, rows_per)],
                              sum_hbm.at[c, pl.ds(r0, rows_per)],
                              ssem[0]).wait()

    return agg


def _dense_body(relu, logsm):
    def body(s_ref, c_ref, xt_ref, wl_ref, wr_ref, b_ref, o_ref):
        ssum = s_ref[0] + s_ref[1]
        cnt_bcast = lax.dot_general(
            c_ref[...], jnp.ones((NC, D), jnp.float32),
            (((0,), (0,)), ((), ())),
            preferred_element_type=jnp.float32,
            precision=lax.Precision.HIGHEST)
        mean = ssum / jnp.maximum(cnt_bcast, 1.0)
        z = lax.dot_general(mean, wl_ref[...], (((1,), (1,)), ((), ())),
                            preferred_element_type=jnp.float32,
                            precision=lax.Precision.HIGHEST)
        z += lax.dot_general(xt_ref[...], wr_ref[...], (((1,), (1,)), ((), ())),
                             preferred_element_type=jnp.float32,
                             precision=lax.Precision.HIGHEST)
        z += b_ref[...]
        if relu:
            z = jnp.maximum(z, 0.0)
        if logsm:
            m = jnp.max(z, axis=-1, keepdims=True)
            z = z - m - jnp.log(jnp.sum(jnp.exp(z - m), axis=-1, keepdims=True))
        o_ref[...] = z
    return body


def _dense(relu, logsm, n_tgt, sums, cnts, xt, wl, wr, b):
    ntp = n_tgt + PAD
    return pl.pallas_call(
        _dense_body(relu, logsm),
        in_specs=[pl.BlockSpec((NC, n_tgt, D), lambda i: (0, 0, 0)),
                  pl.BlockSpec((NC, n_tgt), lambda i: (0, 0)),
                  pl.BlockSpec((n_tgt, D), lambda i: (0, 0)),
                  pl.BlockSpec((D, D), lambda i: (0, 0)),
                  pl.BlockSpec((D, D), lambda i: (0, 0)),
                  pl.BlockSpec((1, D), lambda i: (0, 0))],
        out_specs=pl.BlockSpec((n_tgt, D), lambda i: (0, 0)),
        out_shape=jax.ShapeDtypeStruct((n_tgt, D), jnp.float32),
        grid=(1,),
    )(sums, cnts.reshape(NC, ntp), xt, wl, wr, b.reshape(1, D))


_agg0 = _make_sc_agg(N1, NCH0)
_agg1 = _make_sc_agg(N2, NCH1)

def _pad_edges(src, dst, n_edges_pad, sentinel):
    pad = n_edges_pad - src.shape[0]
    if pad:
        i = jnp.arange(pad, dtype=jnp.int32)
        src = jnp.concatenate([src, i % jnp.int32(sentinel)])
        dst = jnp.concatenate([dst, sentinel + i % jnp.int32(PAD)])
    return src, dst.reshape(-1, C)


def kernel(x, edge_index0, edge_index1, W_l0, W_r0, b0, W_l1, W_r1, b1):
    src0 = edge_index0[0].astype(jnp.int32)
    dst0 = edge_index0[1].astype(jnp.int32)
    src1 = edge_index1[0].astype(jnp.int32)
    dst1 = edge_index1[1].astype(jnp.int32)

    src0, dst0 = _pad_edges(src0, dst0, NW * NCH0 * C, N1)
    src1, dst1 = _pad_edges(src1, dst1, NW * NCH1 * C, N2)
    ones_c = jnp.ones((C,), jnp.float32)

    zs0 = jnp.zeros((N1 + PAD, D), jnp.float32)
    zc0 = jnp.zeros((N1 + PAD,), jnp.float32)
    sums0, cnts0 = _agg0(x, src0, dst0, zs0, zc0, ones_c)
    h = _dense(True, False, N1, sums0, cnts0, x, W_l0, W_r0, b0)

    zs1 = jnp.zeros((N2 + PAD, D), jnp.float32)
    zc1 = jnp.zeros((N2 + PAD,), jnp.float32)
    sums1, cnts1 = _agg1(h, src1, dst1, zs1, zc1, ones_c)
    out = _dense(False, True, N2, sums1, cnts1, h, W_l1, W_r1, b1)
    return out


# speedup vs baseline: 1.0617x; 1.0036x over previous
import dataclasses
import functools

import jax
import jax.numpy as jnp
from jax import lax
from jax.experimental import pallas as pl
from jax.experimental.pallas import tpu as pltpu
from jax.experimental.pallas import tpu_sc as plsc

N0, N1, N2 = 10000, 4096, 1024
E0, E1 = 320000, 131072
D = 128
NC, NS = 2, 16
NW = NC * NS
C = 128
NB = 4
PAD = 128

NCH0 = 80
NCH1 = 32


def _make_sc_agg(n_tgt, n_chunks):
    ntp = n_tgt + PAD
    per_w = n_chunks * C
    rows_per = ntp // NS
    assert rows_per * NS == ntp and rows_per % 8 == 0
    assert n_chunks % NB == 0 and n_chunks >= 2 * NB
    mesh = plsc.VectorSubcoreMesh(core_axis_name="c", subcore_axis_name="s")
    cp = pltpu.CompilerParams()
    if "needs_layout_passes" in pltpu.CompilerParams.__dataclass_fields__:
        cp = dataclasses.replace(cp, needs_layout_passes=False)

    @functools.partial(
        pl.kernel,
        compiler_params=cp,
        out_type=(jax.ShapeDtypeStruct((NC, ntp, D), jnp.float32),
                  jax.ShapeDtypeStruct((NC * ntp,), jnp.float32)),
        mesh=mesh,
        scratch_types=[
            pltpu.VMEM((per_w,), jnp.int32),
            pltpu.VMEM((n_chunks, C), jnp.int32),
            pltpu.VMEM((C, D), jnp.float32),
            pltpu.VMEM((C, D), jnp.float32),
            pltpu.VMEM((C, D), jnp.float32),
            pltpu.VMEM((C, D), jnp.float32),
            pltpu.VMEM((C,), jnp.float32),
            pltpu.VMEM((rows_per,), jnp.float32),
            pltpu.VMEM_SHARED((ntp, D), jnp.float32),
            pltpu.VMEM_SHARED((ntp,), jnp.float32),
        ] + [pltpu.SemaphoreType.DMA] * (3 * NB),
    )
    def agg(x_hbm, src_hbm, dst_hbm, zs_hbm, zc_hbm, on_hbm,
            sum_hbm, cnt_hbm,
            src_v, dst_v, r0_v, r1_v, r2_v, r3_v, on_v, zst_v, acc_sh, cnt_sh,
            *sems):
        rows = (r0_v, r1_v, r2_v, r3_v)
        gsem = sems[0:NB]
        ssem = sems[NB:2 * NB]
        csem = sems[2 * NB:3 * NB]
        c = lax.axis_index("c")
        s = lax.axis_index("s")
        wid = c * NS + s

        def issue_gather(j, b):
            pltpu.async_copy(x_hbm.at[src_v.at[pl.ds(j * C, C)]],
                             rows[b], gsem[b])

        def wait_gather(j, b):
            pltpu.make_async_copy(x_hbm.at[src_v.at[pl.ds(j * C, C)]],
                                  rows[b], gsem[b]).wait()

        r0 = s * rows_per
        pltpu.async_copy(src_hbm.at[pl.ds(wid * per_w, per_w)], src_v,
                         gsem[0])
        pltpu.async_copy(dst_hbm.at[pl.ds(wid * n_chunks, n_chunks)], dst_v,
                         gsem[1])
        pltpu.async_copy(on_hbm, on_v, csem[0])
        pltpu.async_copy(zs_hbm.at[pl.ds(r0, rows_per)],
                         acc_sh.at[pl.ds(r0, rows_per)], ssem[0])
        pltpu.make_async_copy(src_hbm.at[pl.ds(wid * per_w, per_w)], src_v,
                              gsem[0]).wait()
        issue_gather(0, 0)
        issue_gather(1, 1)
        pltpu.make_async_copy(dst_hbm.at[pl.ds(wid * n_chunks, n_chunks)],
                              dst_v, gsem[1]).wait()
        pltpu.make_async_copy(on_hbm, on_v, csem[0]).wait()
        pltpu.make_async_copy(zs_hbm.at[pl.ds(r0, rows_per)],
                              acc_sh.at[pl.ds(r0, rows_per)], ssem[0]).wait()
        pltpu.sync_copy(zc_hbm.at[pl.ds(r0, rows_per)], zst_v)
        pltpu.sync_copy(zst_v, cnt_sh.at[pl.ds(r0, rows_per)])
        plsc.subcore_barrier()

        def issue_scatter(j, b):
            pltpu.async_copy(rows[b], acc_sh.at[dst_v.at[j]], ssem[b],
                             add=True)
            pltpu.async_copy(on_v, cnt_sh.at[dst_v.at[j]], csem[b],
                             add=True)

        def wait_scatter(j, b):
            pltpu.make_async_copy(rows[b], acc_sh.at[dst_v.at[j]],
                                  ssem[b]).wait()
            pltpu.make_async_copy(on_v, cnt_sh.at[dst_v.at[j]],
                                  csem[b]).wait()


        for b in range(NB):
            j = b
            if j >= 2:
                wait_scatter(j - 2, (b + 2) % NB)
            issue_gather(j + 2, (b + 2) % NB)
            wait_gather(j, b)
            issue_scatter(j, b)

        @pl.loop(1, n_chunks // NB - 1)
        def _(k):
            j0 = k * NB
            for b in range(NB):
                j = j0 + b
                wait_scatter(j - 2, (b + 2) % NB)
                issue_gather(j + 2, (b + 2) % NB)
                wait_gather(j, b)
                issue_scatter(j, b)

        for b in range(NB):
            j = n_chunks - NB + b
            if b < 2:
                wait_scatter(j - 2, (b + 2) % NB)
                issue_gather(j + 2, (b + 2) % NB)
            wait_gather(j, b)
            issue_scatter(j, b)
        for b in range(NB):
            wait_scatter(n_chunks - NB + b, b)

        plsc.subcore_barrier()
        pltpu.async_copy(acc_sh.at[pl.ds(r0, rows_per)],
                         sum_hbm.at[c, pl.ds(r0, rows_per)], ssem[0])
        pltpu.sync_copy(cnt_sh.at[pl.ds(r0, rows_per)], zst_v)
        pltpu.sync_copy(zst_v, cnt_hbm.at[pl.ds(c * ntp + r0, rows_per)])
        pltpu.make_async_copy(acc_sh.at[pl.ds(r0, rows_per)],
                              sum_hbm.at[c, pl.ds(r0, rows_per)],
                              ssem[0]).wait()

    return agg


def _dense_body(relu, logsm):
    def body(s_ref, c_ref, xt_ref, wl_ref, wr_ref, b_ref, o_ref):
        ssum = s_ref[0] + s_ref[1]
        cnt_bcast = lax.dot_general(
            c_ref[...], jnp.ones((NC, D), jnp.float32),
            (((0,), (0,)), ((), ())),
            preferred_element_type=jnp.float32,
            precision=lax.Precision.HIGHEST)
        mean = ssum / jnp.maximum(cnt_bcast, 1.0)

        def mm_t(a, w):
            a_hi = a.astype(jnp.bfloat16)
            a_lo = (a - a_hi.astype(jnp.float32)).astype(jnp.bfloat16)
            w_hi = w.astype(jnp.bfloat16)
            w_lo = (w - w_hi.astype(jnp.float32)).astype(jnp.bfloat16)
            dn = (((1,), (1,)), ((), ()))
            out = lax.dot_general(a_hi, w_hi, dn,
                                  preferred_element_type=jnp.float32)
            out += lax.dot_general(a_lo, w_hi, dn,
                                   preferred_element_type=jnp.float32)
            out += lax.dot_general(a_hi, w_lo, dn,
                                   preferred_element_type=jnp.float32)
            return out

        z = mm_t(mean, wl_ref[...]) + mm_t(xt_ref[...], wr_ref[...])
        z += b_ref[...]
        if relu:
            z = jnp.maximum(z, 0.0)
        if logsm:
            m = jnp.max(z, axis=-1, keepdims=True)
            z = z - m - jnp.log(jnp.sum(jnp.exp(z - m), axis=-1, keepdims=True))
        o_ref[...] = z
    return body


def _dense(relu, logsm, n_tgt, sums, cnts, xt, wl, wr, b):
    ntp = n_tgt + PAD
    return pl.pallas_call(
        _dense_body(relu, logsm),
        in_specs=[pl.BlockSpec((NC, n_tgt, D), lambda i: (0, 0, 0)),
                  pl.BlockSpec((NC, n_tgt), lambda i: (0, 0)),
                  pl.BlockSpec((n_tgt, D), lambda i: (0, 0)),
                  pl.BlockSpec((D, D), lambda i: (0, 0)),
                  pl.BlockSpec((D, D), lambda i: (0, 0)),
                  pl.BlockSpec((1, D), lambda i: (0, 0))],
        out_specs=pl.BlockSpec((n_tgt, D), lambda i: (0, 0)),
        out_shape=jax.ShapeDtypeStruct((n_tgt, D), jnp.float32),
        grid=(1,),
    )(sums, cnts.reshape(NC, ntp), xt, wl, wr, b.reshape(1, D))


_agg0 = _make_sc_agg(N1, NCH0)
_agg1 = _make_sc_agg(N2, NCH1)

def _pad_edges(src, dst, n_edges_pad, sentinel):
    pad = n_edges_pad - src.shape[0]
    if pad:
        i = jnp.arange(pad, dtype=jnp.int32)
        src = jnp.concatenate([src, i % jnp.int32(sentinel)])
        dst = jnp.concatenate([dst, sentinel + i % jnp.int32(PAD)])
    return src, dst.reshape(-1, C)


def kernel(x, edge_index0, edge_index1, W_l0, W_r0, b0, W_l1, W_r1, b1):
    src0 = edge_index0[0].astype(jnp.int32)
    dst0 = edge_index0[1].astype(jnp.int32)
    src1 = edge_index1[0].astype(jnp.int32)
    dst1 = edge_index1[1].astype(jnp.int32)

    src0, dst0 = _pad_edges(src0, dst0, NW * NCH0 * C, N1)
    src1, dst1 = _pad_edges(src1, dst1, NW * NCH1 * C, N2)
    ones_c = jnp.ones((C,), jnp.float32)

    zs0 = jnp.zeros((N1 + PAD, D), jnp.float32)
    zc0 = jnp.zeros((N1 + PAD,), jnp.float32)
    sums0, cnts0 = _agg0(x, src0, dst0, zs0, zc0, ones_c)
    h = _dense(True, False, N1, sums0, cnts0, x, W_l0, W_r0, b0)

    zs1 = jnp.zeros((N2 + PAD, D), jnp.float32)
    zc1 = jnp.zeros((N2 + PAD,), jnp.float32)
    sums1, cnts1 = _agg1(h, src1, dst1, zs1, zc1, ones_c)
    out = _dense(False, True, N2, sums1, cnts1, h, W_l1, W_r1, b1)
    return out
